# baseline (speedup 1.0000x reference)
import functools

import jax
import jax.numpy as jnp
from jax import lax
from jax.experimental import pallas as pl
from jax.experimental.pallas import tpu as pltpu
from jax.experimental.pallas import tpu_sc as plsc

L = 2048
NB = 4
ROWS = 2 * NB * L
NW = 32
SEG = ROWS // (2 * NW)
CHUNK = 16
LANES = 16


def _sc_call(seq):
    mesh = plsc.VectorSubcoreMesh(core_axis_name="c", subcore_axis_name="s")

    @functools.partial(
        pl.kernel,
        mesh=mesh,
        out_type=jax.ShapeDtypeStruct((ROWS, L), jnp.float32),
        scratch_types=[
            pltpu.VMEM((L,), jnp.int32),
            pltpu.VMEM((CHUNK, L), jnp.float32),
            pltpu.VMEM((2, L), jnp.float32),
            pltpu.SemaphoreType.DMA,
            pltpu.SemaphoreType.DMA,
        ],
    )
    def k(seq_hbm, out_hbm, seq_v, pat, tab, sem_b, sem_t):
        nc = 2
        wid = lax.axis_index("s") * nc + lax.axis_index("c")
        sym = wid // (NW // NB)
        i0 = wid * SEG - sym * L
        top_base = wid * SEG
        bot_base = NB * L + wid * SEG

        pltpu.sync_copy(seq_hbm, seq_v)

        def tbuild(j, _):
            tab[0, pl.ds(j * LANES, LANES)] = jnp.zeros((LANES,), jnp.float32)
            tab[1, pl.ds(j * LANES, LANES)] = jnp.ones((LANES,), jnp.float32)
            return 0
        lax.fori_loop(0, L // LANES, tbuild, 0)

        def dt(mb, _):
            v16 = seq_v[pl.ds(i0 + mb * LANES, LANES)]
            bvec = jnp.where(v16 == sym, 1, 0).astype(jnp.int32)
            for kk in range(LANES):
                pltpu.async_copy(
                    tab.at[pl.ds(bvec[kk], 1)],
                    out_hbm.at[pl.ds(top_base + mb * LANES + kk, 1)], sem_t)
            return 0
        lax.fori_loop(0, SEG // LANES, dt, 0)

        def pbuild(j, _):
            v = jnp.where(
                seq_v[pl.ds(j * LANES, LANES)] == sym, 1.0, 0.0
            ).astype(jnp.float32)
            for kk in range(CHUNK):
                pat[kk, pl.ds(j * LANES, LANES)] = v
            return 0
        lax.fori_loop(0, L // LANES, pbuild, 0)

        def db(m, _):
            pltpu.async_copy(
                pat, out_hbm.at[pl.ds(bot_base + m * CHUNK, CHUNK)], sem_b)
            return 0
        lax.fori_loop(0, SEG // CHUNK, db, 0)

        def wt(m, _):
            pltpu.make_async_copy(
                tab.at[pl.ds(0, 1)],
                out_hbm.at[pl.ds(top_base, 1)], sem_t).wait()
            return 0
        lax.fori_loop(0, SEG, wt, 0)

        def wb(m, _):
            pltpu.make_async_copy(
                pat, out_hbm.at[pl.ds(bot_base, CHUNK)], sem_b).wait()
            return 0
        lax.fori_loop(0, SEG // CHUNK, wb, 0)

    return k(seq)


def kernel(sequence):
    seq = sequence.astype(jnp.int32)
    out = _sc_call(seq)
    return out.reshape(2 * NB, L, L)
